# Initial kernel scaffold; baseline (speedup 1.0000x reference)
import jax
import jax.numpy as jnp
from jax.experimental import pallas as pl


def kernel(batch_size, num_views, coordinates, angles, timestamps, x_emb, y_emb, z_emb, az_emb, el_emb, t_emb):
    raise NotImplementedError("write your pallas kernel here")


# speedup vs baseline: 1.4575x; 1.4575x over previous
import jax
import jax.numpy as jnp
from jax import lax
from jax.experimental import pallas as pl
from jax.experimental.pallas import tpu as pltpu
from jax.experimental.pallas import tpu_sc as plsc

NC, NS = 2, 16
NW = NC * NS
COORD_DIM = 85
SPATIAL_DIM = 256
ROW_DIM = 3 * COORD_DIM + 3 * SPATIAL_DIM
CHUNK = 32


def _assemble_segment(abuf, buf, r, offset, width, iota):
    dst = r * ROW_DIM + offset + iota
    nfull = width // 16
    for i in range(nfull):
        v = buf[r, pl.ds(16 * i, 16)]
        plsc.store_scatter(abuf, [dst + 16 * i], v)
    rem = width - 16 * nfull
    if rem:
        cols = 16 * nfull + iota
        m = cols < width
        rows = lax.broadcast(r, (16,))
        v = plsc.load_gather(buf, [rows, cols], mask=m)
        plsc.store_scatter(abuf, [dst + 16 * nfull], v, mask=m)


def _body(ix, iy, iz, ia, ie, it, x_tab, y_tab, z_tab, a_tab, e_tab, t_tab,
          out, vix, viy, viz, via, vie, vit, bx, by, bz, ba, be, bt, abuf,
          sem):
    wid = lax.axis_index("s") * NC + lax.axis_index("c")
    rpw = vix.shape[0]
    base = wid * rpw
    pltpu.sync_copy(ix.at[pl.ds(base, rpw)], vix)
    pltpu.sync_copy(iy.at[pl.ds(base, rpw)], viy)
    pltpu.sync_copy(iz.at[pl.ds(base, rpw)], viz)
    pltpu.sync_copy(ia.at[pl.ds(base, rpw)], via)
    pltpu.sync_copy(ie.at[pl.ds(base, rpw)], vie)
    pltpu.sync_copy(it.at[pl.ds(base, rpw)], vit)
    iota = lax.iota(jnp.int32, 16)

    @pl.loop(0, rpw // CHUNK)
    def _chunk(c):
        off = c * CHUNK
        d0 = pltpu.async_copy(x_tab.at[vix.at[pl.ds(off, CHUNK)]], bx, sem)
        d1 = pltpu.async_copy(y_tab.at[viy.at[pl.ds(off, CHUNK)]], by, sem)
        d2 = pltpu.async_copy(z_tab.at[viz.at[pl.ds(off, CHUNK)]], bz, sem)
        d3 = pltpu.async_copy(a_tab.at[via.at[pl.ds(off, CHUNK)]], ba, sem)
        d4 = pltpu.async_copy(e_tab.at[vie.at[pl.ds(off, CHUNK)]], be, sem)
        d5 = pltpu.async_copy(t_tab.at[vit.at[pl.ds(off, CHUNK)]], bt, sem)
        d0.wait(); d1.wait(); d2.wait(); d3.wait(); d4.wait(); d5.wait()

        @pl.loop(0, CHUNK)
        def _row(r):
            _assemble_segment(abuf, bx, r, 0, COORD_DIM, iota)
            _assemble_segment(abuf, by, r, 85, COORD_DIM, iota)
            _assemble_segment(abuf, bz, r, 170, COORD_DIM, iota)
            _assemble_segment(abuf, ba, r, 255, SPATIAL_DIM, iota)
            _assemble_segment(abuf, be, r, 511, SPATIAL_DIM, iota)
            _assemble_segment(abuf, bt, r, 767, SPATIAL_DIM, iota)

        pltpu.sync_copy(
            abuf, out.at[pl.ds((base + off) * ROW_DIM, CHUNK * ROW_DIM)])


def kernel(batch_size, num_views, coordinates, angles, timestamps,
           x_emb, y_emb, z_emb, az_emb, el_emb, t_emb):
    b, v = coordinates.shape[0], coordinates.shape[1]
    n = b * v
    rpw = n // NW
    cx = coordinates[..., 0].reshape(-1).astype(jnp.int32)
    cy = coordinates[..., 1].reshape(-1).astype(jnp.int32)
    cz = coordinates[..., 2].reshape(-1).astype(jnp.int32)
    az = angles[..., 0].reshape(-1).astype(jnp.int32)
    el = angles[..., 1].reshape(-1).astype(jnp.int32)
    ts = timestamps.reshape(-1).astype(jnp.int32)
    x_p = jnp.pad(x_emb, ((0, 0), (0, 3)))
    y_p = jnp.pad(y_emb, ((0, 0), (0, 3)))
    z_p = jnp.pad(z_emb, ((0, 0), (0, 3)))

    run = pl.kernel(
        _body,
        out_type=jax.ShapeDtypeStruct((n * ROW_DIM,), jnp.float32),
        mesh=plsc.VectorSubcoreMesh(
            core_axis_name="c", subcore_axis_name="s",
            num_cores=NC, num_subcores=NS),
        compiler_params=pltpu.CompilerParams(
            use_tc_tiling_on_sc=False, needs_layout_passes=False),
        scratch_types=[
            pltpu.VMEM((rpw,), jnp.int32),
            pltpu.VMEM((rpw,), jnp.int32),
            pltpu.VMEM((rpw,), jnp.int32),
            pltpu.VMEM((rpw,), jnp.int32),
            pltpu.VMEM((rpw,), jnp.int32),
            pltpu.VMEM((rpw,), jnp.int32),
            pltpu.VMEM((CHUNK, 88), jnp.float32),
            pltpu.VMEM((CHUNK, 88), jnp.float32),
            pltpu.VMEM((CHUNK, 88), jnp.float32),
            pltpu.VMEM((CHUNK, SPATIAL_DIM), jnp.float32),
            pltpu.VMEM((CHUNK, SPATIAL_DIM), jnp.float32),
            pltpu.VMEM((CHUNK, SPATIAL_DIM), jnp.float32),
            pltpu.VMEM((CHUNK * ROW_DIM,), jnp.float32),
            pltpu.SemaphoreType.DMA,
        ],
    )
    flat = run(cx, cy, cz, az, el, ts, x_p, y_p, z_p, az_emb, el_emb, t_emb)
    return flat.reshape(b, v, ROW_DIM)


# speedup vs baseline: 1.8304x; 1.2558x over previous
import jax
import jax.numpy as jnp
from jax import lax
from jax.experimental import pallas as pl
from jax.experimental.pallas import tpu as pltpu
from jax.experimental.pallas import tpu_sc as plsc

NC, NS = 2, 16
NW = NC * NS
COORD_DIM = 85
SPATIAL_DIM = 256
PAD_DIM = 96
ROW_DIM = 3 * COORD_DIM + 3 * SPATIAL_DIM
CHUNK = 16
SEGS = ((0, 0, PAD_DIM), (1, 85, PAD_DIM), (2, 170, PAD_DIM),
        (3, 255, SPATIAL_DIM), (4, 511, SPATIAL_DIM), (5, 767, SPATIAL_DIM))


def _body(ix, iy, iz, ia, ie, it, x_tab, y_tab, z_tab, a_tab, e_tab, t_tab,
          out, vidx, bufs, abufs, gsems, wsems):
    wid = lax.axis_index("s") * NC + lax.axis_index("c")
    vix, viy, viz, via, vie, vit = vidx
    tabs = (x_tab, y_tab, z_tab, a_tab, e_tab, t_tab)
    rpw = vix.shape[0]
    base = wid * rpw
    nch = rpw // CHUNK
    for hsrc, vdst in zip((ix, iy, iz, ia, ie, it), vidx):
        pltpu.sync_copy(hsrc.at[pl.ds(base, rpw)], vdst)
    iota = lax.iota(jnp.int32, 16)
    consts = [iota + 16 * i for i in range(SPATIAL_DIM // 16)]

    def issue_gathers(p, c):
        off = c * CHUNK
        for k in range(6):
            pltpu.async_copy(tabs[k].at[vidx[k].at[pl.ds(off, CHUNK)]],
                             bufs[p][k], gsems[p])

    def drain_gathers(p):
        for k in range(6):
            pltpu.make_async_copy(tabs[k].at[vidx[k].at[pl.ds(0, CHUNK)]],
                                  bufs[p][k], gsems[p]).wait()

    def assemble(p):
        abuf = abufs[p]

        @pl.loop(0, CHUNK)
        def _row(r):
            rbase = r * ROW_DIM
            for k, o, w in SEGS:
                sb = rbase + o
                buf = bufs[p][k]
                for i in range(w // 16):
                    v = buf[r, pl.ds(16 * i, 16)]
                    plsc.store_scatter(abuf, [sb + consts[i]], v)

    def write_out(p, c):
        pltpu.async_copy(
            abufs[p],
            out.at[pl.ds((base + c * CHUNK) * ROW_DIM, CHUNK * ROW_DIM)],
            wsems[p])

    def wait_write(p):
        pltpu.make_async_copy(
            abufs[p], out.at[pl.ds(0, CHUNK * ROW_DIM)], wsems[p]).wait()

    issue_gathers(0, 0)

    @pl.loop(0, nch, step=2)
    def _pair(c):
        issue_gathers(1, c + 1)
        drain_gathers(0)

        @pl.when(c > 0)
        def _():
            wait_write(0)
        assemble(0)
        write_out(0, c)
        issue_gathers(0, lax.min(c + 2, nch - 1))
        drain_gathers(1)

        @pl.when(c > 0)
        def _():
            wait_write(1)
        assemble(1)
        write_out(1, c + 1)

    drain_gathers(0)
    wait_write(0)
    wait_write(1)


def kernel(batch_size, num_views, coordinates, angles, timestamps,
           x_emb, y_emb, z_emb, az_emb, el_emb, t_emb):
    b, v = coordinates.shape[0], coordinates.shape[1]
    n = b * v
    rpw = n // NW
    cx = coordinates[..., 0].reshape(-1).astype(jnp.int32)
    cy = coordinates[..., 1].reshape(-1).astype(jnp.int32)
    cz = coordinates[..., 2].reshape(-1).astype(jnp.int32)
    az = angles[..., 0].reshape(-1).astype(jnp.int32)
    el = angles[..., 1].reshape(-1).astype(jnp.int32)
    ts = timestamps.reshape(-1).astype(jnp.int32)
    pad = ((0, 0), (0, PAD_DIM - COORD_DIM))
    x_p, y_p, z_p = (jnp.pad(t, pad) for t in (x_emb, y_emb, z_emb))

    bufset = (
        pltpu.VMEM((CHUNK, PAD_DIM), jnp.float32),
        pltpu.VMEM((CHUNK, PAD_DIM), jnp.float32),
        pltpu.VMEM((CHUNK, PAD_DIM), jnp.float32),
        pltpu.VMEM((CHUNK, SPATIAL_DIM), jnp.float32),
        pltpu.VMEM((CHUNK, SPATIAL_DIM), jnp.float32),
        pltpu.VMEM((CHUNK, SPATIAL_DIM), jnp.float32),
    )
    run = pl.kernel(
        _body,
        out_type=jax.ShapeDtypeStruct((n * ROW_DIM,), jnp.float32),
        mesh=plsc.VectorSubcoreMesh(
            core_axis_name="c", subcore_axis_name="s",
            num_cores=NC, num_subcores=NS),
        compiler_params=pltpu.CompilerParams(
            use_tc_tiling_on_sc=False, needs_layout_passes=False),
        scratch_types=[
            tuple(pltpu.VMEM((rpw,), jnp.int32) for _ in range(6)),
            (bufset, bufset),
            tuple(pltpu.VMEM((CHUNK * ROW_DIM,), jnp.float32)
                  for _ in range(2)),
            (pltpu.SemaphoreType.DMA, pltpu.SemaphoreType.DMA),
            (pltpu.SemaphoreType.DMA, pltpu.SemaphoreType.DMA),
        ],
    )
    flat = run(cx, cy, cz, az, el, ts, x_p, y_p, z_p, az_emb, el_emb, t_emb)
    return flat.reshape(b, v, ROW_DIM)


# speedup vs baseline: 2.4187x; 1.3214x over previous
import jax
import jax.numpy as jnp
from jax import lax
from jax.experimental import pallas as pl
from jax.experimental.pallas import tpu as pltpu
from jax.experimental.pallas import tpu_sc as plsc

NC, NS = 2, 16
NW = NC * NS
COORD_DIM = 85
SPATIAL_DIM = 256
PAD_DIM = 96
ROW_DIM = 3 * COORD_DIM + 3 * SPATIAL_DIM
CHUNK = 16
SEGS = ((0, 0, PAD_DIM), (1, 85, PAD_DIM), (2, 170, PAD_DIM),
        (3, 255, SPATIAL_DIM), (4, 511, SPATIAL_DIM), (5, 767, SPATIAL_DIM))


def _body(ix, iy, iz, ia, ie, it, x_tab, y_tab, z_tab, a_tab, e_tab, t_tab,
          out, vidx, bufs, abufs, gsems, wsems):
    wid = lax.axis_index("s") * NC + lax.axis_index("c")
    vix, viy, viz, via, vie, vit = vidx
    tabs = (x_tab, y_tab, z_tab, a_tab, e_tab, t_tab)
    rpw = vix.shape[0]
    base = wid * rpw
    nch = rpw // CHUNK
    for hsrc, vdst in zip((ix, iy, iz, ia, ie, it), vidx):
        pltpu.sync_copy(hsrc.at[pl.ds(base, rpw)], vdst)
    iota = lax.iota(jnp.int32, 16)
    consts = [iota + 16 * i for i in range(SPATIAL_DIM // 16)]

    def issue_gathers(p, c):
        off = c * CHUNK
        for k in range(6):
            pltpu.async_copy(tabs[k].at[vidx[k].at[pl.ds(off, CHUNK)]],
                             bufs[p][k], gsems[p])

    def drain_gathers(p):
        for k in range(6):
            pltpu.make_async_copy(tabs[k].at[vidx[k].at[pl.ds(0, CHUNK)]],
                                  bufs[p][k], gsems[p]).wait()

    def assemble(p):
        abuf = abufs[p]

        @pl.loop(0, CHUNK)
        def _row(r):
            rbase = r * ROW_DIM
            ops = []
            for k, o, w in SEGS:
                sbv = lax.broadcast(rbase + o, (16,))
                buf = bufs[p][k]
                for i in range(w // 16):
                    ops.append((buf, 16 * i, sbv + consts[i]))
            for g in range(0, len(ops), 8):
                grp = ops[g:g + 8]
                vs = [buf[r, pl.ds(c0, 16)] for buf, c0, _ in grp]
                for (_, _, dst), v in zip(grp, vs):
                    plsc.store_scatter(abuf, [dst], v)

    def write_out(p, c):
        pltpu.async_copy(
            abufs[p],
            out.at[pl.ds((base + c * CHUNK) * ROW_DIM, CHUNK * ROW_DIM)],
            wsems[p])

    def wait_write(p):
        pltpu.make_async_copy(
            abufs[p], out.at[pl.ds(0, CHUNK * ROW_DIM)], wsems[p]).wait()

    issue_gathers(0, 0)

    @pl.loop(0, nch, step=2)
    def _pair(c):
        issue_gathers(1, c + 1)
        drain_gathers(0)

        @pl.when(c > 0)
        def _():
            wait_write(0)
        assemble(0)
        write_out(0, c)
        issue_gathers(0, lax.min(c + 2, nch - 1))
        drain_gathers(1)

        @pl.when(c > 0)
        def _():
            wait_write(1)
        assemble(1)
        write_out(1, c + 1)

    drain_gathers(0)
    wait_write(0)
    wait_write(1)


def kernel(batch_size, num_views, coordinates, angles, timestamps,
           x_emb, y_emb, z_emb, az_emb, el_emb, t_emb):
    b, v = coordinates.shape[0], coordinates.shape[1]
    n = b * v
    rpw = n // NW
    cx = coordinates[..., 0].reshape(-1).astype(jnp.int32)
    cy = coordinates[..., 1].reshape(-1).astype(jnp.int32)
    cz = coordinates[..., 2].reshape(-1).astype(jnp.int32)
    az = angles[..., 0].reshape(-1).astype(jnp.int32)
    el = angles[..., 1].reshape(-1).astype(jnp.int32)
    ts = timestamps.reshape(-1).astype(jnp.int32)
    pad = ((0, 0), (0, PAD_DIM - COORD_DIM))
    x_p, y_p, z_p = (jnp.pad(t, pad) for t in (x_emb, y_emb, z_emb))

    bufset = (
        pltpu.VMEM((CHUNK, PAD_DIM), jnp.float32),
        pltpu.VMEM((CHUNK, PAD_DIM), jnp.float32),
        pltpu.VMEM((CHUNK, PAD_DIM), jnp.float32),
        pltpu.VMEM((CHUNK, SPATIAL_DIM), jnp.float32),
        pltpu.VMEM((CHUNK, SPATIAL_DIM), jnp.float32),
        pltpu.VMEM((CHUNK, SPATIAL_DIM), jnp.float32),
    )
    run = pl.kernel(
        _body,
        out_type=jax.ShapeDtypeStruct((n * ROW_DIM,), jnp.float32),
        mesh=plsc.VectorSubcoreMesh(
            core_axis_name="c", subcore_axis_name="s",
            num_cores=NC, num_subcores=NS),
        compiler_params=pltpu.CompilerParams(
            use_tc_tiling_on_sc=False, needs_layout_passes=False),
        scratch_types=[
            tuple(pltpu.VMEM((rpw,), jnp.int32) for _ in range(6)),
            (bufset, bufset),
            tuple(pltpu.VMEM((CHUNK * ROW_DIM,), jnp.float32)
                  for _ in range(2)),
            (pltpu.SemaphoreType.DMA, pltpu.SemaphoreType.DMA),
            (pltpu.SemaphoreType.DMA, pltpu.SemaphoreType.DMA),
        ],
    )
    flat = run(cx, cy, cz, az, el, ts, x_p, y_p, z_p, az_emb, el_emb, t_emb)
    return flat.reshape(b, v, ROW_DIM)


# speedup vs baseline: 2.4372x; 1.0077x over previous
import jax
import jax.numpy as jnp
from jax import lax
from jax.experimental import pallas as pl
from jax.experimental.pallas import tpu as pltpu
from jax.experimental.pallas import tpu_sc as plsc

NC, NS = 2, 16
NW = NC * NS
COORD_DIM = 85
SPATIAL_DIM = 256
PAD_DIM = 96
ROW_DIM = 3 * COORD_DIM + 3 * SPATIAL_DIM
CHUNK = 16
SEGS = ((0, 0, PAD_DIM), (1, 85, PAD_DIM), (2, 170, PAD_DIM),
        (3, 255, SPATIAL_DIM), (4, 511, SPATIAL_DIM), (5, 767, SPATIAL_DIM))


def _body(ix, iy, iz, ia, ie, it, x_tab, y_tab, z_tab, a_tab, e_tab, t_tab,
          out, vidx, bufs, abufs, gsems, wsems):
    wid = lax.axis_index("s") * NC + lax.axis_index("c")
    vix, viy, viz, via, vie, vit = vidx
    tabs = (x_tab, y_tab, z_tab, a_tab, e_tab, t_tab)
    rpw = vix.shape[0]
    base = wid * rpw
    nch = rpw // CHUNK
    for hsrc, vdst in zip((ix, iy, iz, ia, ie, it), vidx):
        pltpu.sync_copy(hsrc.at[pl.ds(base, rpw)], vdst)
    iota = lax.iota(jnp.int32, 16)
    consts = [iota + 16 * i for i in range(SPATIAL_DIM // 16)]

    def issue_gathers(p, c):
        off = c * CHUNK
        for k in range(6):
            pltpu.async_copy(tabs[k].at[vidx[k].at[pl.ds(off, CHUNK)]],
                             bufs[p][k], gsems[p])

    def drain_gathers(p):
        for k in range(6):
            pltpu.make_async_copy(tabs[k].at[vidx[k].at[pl.ds(0, CHUNK)]],
                                  bufs[p][k], gsems[p]).wait()

    def assemble(p):
        abuf = abufs[p]

        @plsc.parallel_loop(0, CHUNK, 1, unroll=1)
        def _row(r):
            rbase = r * ROW_DIM
            ops = []
            for k, o, w in SEGS:
                sbv = lax.broadcast(rbase + o, (16,))
                buf = bufs[p][k]
                for i in range(w // 16):
                    ops.append((buf, 16 * i, sbv + consts[i]))
            for g in range(0, len(ops), 8):
                grp = ops[g:g + 8]
                vs = [buf[r, pl.ds(c0, 16)] for buf, c0, _ in grp]
                for (_, _, dst), v in zip(grp, vs):
                    plsc.store_scatter(abuf, [dst], v)

    def write_out(p, c):
        pltpu.async_copy(
            abufs[p],
            out.at[pl.ds((base + c * CHUNK) * ROW_DIM, CHUNK * ROW_DIM)],
            wsems[p])

    def wait_write(p):
        pltpu.make_async_copy(
            abufs[p], out.at[pl.ds(0, CHUNK * ROW_DIM)], wsems[p]).wait()

    issue_gathers(0, 0)

    @pl.loop(0, nch, step=2)
    def _pair(c):
        issue_gathers(1, c + 1)
        drain_gathers(0)

        @pl.when(c > 0)
        def _():
            wait_write(0)
        assemble(0)
        write_out(0, c)
        issue_gathers(0, lax.min(c + 2, nch - 1))
        drain_gathers(1)

        @pl.when(c > 0)
        def _():
            wait_write(1)
        assemble(1)
        write_out(1, c + 1)

    drain_gathers(0)
    wait_write(0)
    wait_write(1)


def kernel(batch_size, num_views, coordinates, angles, timestamps,
           x_emb, y_emb, z_emb, az_emb, el_emb, t_emb):
    b, v = coordinates.shape[0], coordinates.shape[1]
    n = b * v
    rpw = n // NW
    cx = coordinates[..., 0].reshape(-1).astype(jnp.int32)
    cy = coordinates[..., 1].reshape(-1).astype(jnp.int32)
    cz = coordinates[..., 2].reshape(-1).astype(jnp.int32)
    az = angles[..., 0].reshape(-1).astype(jnp.int32)
    el = angles[..., 1].reshape(-1).astype(jnp.int32)
    ts = timestamps.reshape(-1).astype(jnp.int32)
    pad = ((0, 0), (0, PAD_DIM - COORD_DIM))
    x_p, y_p, z_p = (jnp.pad(t, pad) for t in (x_emb, y_emb, z_emb))

    bufset = (
        pltpu.VMEM((CHUNK, PAD_DIM), jnp.float32),
        pltpu.VMEM((CHUNK, PAD_DIM), jnp.float32),
        pltpu.VMEM((CHUNK, PAD_DIM), jnp.float32),
        pltpu.VMEM((CHUNK, SPATIAL_DIM), jnp.float32),
        pltpu.VMEM((CHUNK, SPATIAL_DIM), jnp.float32),
        pltpu.VMEM((CHUNK, SPATIAL_DIM), jnp.float32),
    )
    run = pl.kernel(
        _body,
        out_type=jax.ShapeDtypeStruct((n * ROW_DIM,), jnp.float32),
        mesh=plsc.VectorSubcoreMesh(
            core_axis_name="c", subcore_axis_name="s",
            num_cores=NC, num_subcores=NS),
        compiler_params=pltpu.CompilerParams(
            use_tc_tiling_on_sc=False, needs_layout_passes=False),
        scratch_types=[
            tuple(pltpu.VMEM((rpw,), jnp.int32) for _ in range(6)),
            (bufset, bufset),
            tuple(pltpu.VMEM((CHUNK * ROW_DIM,), jnp.float32)
                  for _ in range(2)),
            (pltpu.SemaphoreType.DMA, pltpu.SemaphoreType.DMA),
            (pltpu.SemaphoreType.DMA, pltpu.SemaphoreType.DMA),
        ],
    )
    flat = run(cx, cy, cz, az, el, ts, x_p, y_p, z_p, az_emb, el_emb, t_emb)
    return flat.reshape(b, v, ROW_DIM)
